# baseline (speedup 1.0000x reference)
import functools

import jax
import jax.numpy as jnp
from jax import lax
from jax.experimental import pallas as pl
from jax.experimental.pallas import tpu as pltpu
from jax.experimental.pallas import tpu_sc as plsc

_BATCH = 16384
_DIM = 64
_LANES = 16
_NUM_CORES = 2
_NUM_SUBCORES = 16
_NW = _NUM_CORES * _NUM_SUBCORES
_BPW = _BATCH // _NW
_CH = 128
_NCHUNK = _BPW // _CH


def _sc_body(h_hbm, r_hbm, t_hbm, ent_hbm, rel_hbm, out_hbm,
             hidx, ridx, tidx, hbuf, rbuf, tbuf, sem):
    wid = lax.axis_index("s") * _NUM_CORES + lax.axis_index("c")
    base = wid * _BPW

    for j in range(_NCHUNK):
        off = base + j * _CH
        pltpu.sync_copy(h_hbm.at[pl.ds(off, _CH)], hidx.at[j])
        pltpu.sync_copy(r_hbm.at[pl.ds(off, _CH)], ridx.at[j])
        pltpu.sync_copy(t_hbm.at[pl.ds(off, _CH)], tidx.at[j])

    for j in range(_NCHUNK):
        ch = pltpu.async_copy(ent_hbm.at[hidx.at[j]], hbuf, sem)
        ct = pltpu.async_copy(ent_hbm.at[tidx.at[j]], tbuf, sem)
        cr = pltpu.async_copy(rel_hbm.at[ridx.at[j]], rbuf, sem)
        ch.wait()
        ct.wait()
        cr.wait()

        def row(i, _):
            for c in range(_DIM // _LANES):
                sl = pl.ds(c * _LANES, _LANES)
                hbuf[i, sl] = hbuf[i, sl] + rbuf[i, sl] - tbuf[i, sl]
            return 0

        lax.fori_loop(0, _CH, row, 0)
        pltpu.sync_copy(hbuf, out_hbm.at[pl.ds(base + j * _CH, _CH)])


_trans_e = functools.partial(
    pl.kernel,
    mesh=plsc.VectorSubcoreMesh(core_axis_name="c", subcore_axis_name="s"),
    out_type=jax.ShapeDtypeStruct((_BATCH, _DIM), jnp.float32),
    scratch_types=[
        pltpu.VMEM((_NCHUNK, _CH), jnp.int32),
        pltpu.VMEM((_NCHUNK, _CH), jnp.int32),
        pltpu.VMEM((_NCHUNK, _CH), jnp.int32),
        pltpu.VMEM((_CH, _DIM), jnp.float32),
        pltpu.VMEM((_CH, _DIM), jnp.float32),
        pltpu.VMEM((_CH, _DIM), jnp.float32),
        pltpu.SemaphoreType.DMA,
    ],
    compiler_params=pltpu.CompilerParams(use_tc_tiling_on_sc=False),
)(_sc_body)


@jax.jit
def kernel(h, r, t, entity_embeddings, relation_embeddings):
    return _trans_e(
        h.astype(jnp.int32),
        r.astype(jnp.int32),
        t.astype(jnp.int32),
        entity_embeddings,
        relation_embeddings,
    )


# speedup vs baseline: 1.0054x; 1.0054x over previous
import functools

import jax
import jax.numpy as jnp
from jax import lax
from jax.experimental import pallas as pl
from jax.experimental.pallas import tpu as pltpu
from jax.experimental.pallas import tpu_sc as plsc

_BATCH = 16384
_DIM = 64
_LANES = 16
_NUM_CORES = 2
_NUM_SUBCORES = 16
_NW = _NUM_CORES * _NUM_SUBCORES
_BPW = _BATCH // _NW
_CH = 128
_NCHUNK = _BPW // _CH


def _sc_body(h_hbm, r_hbm, t_hbm, ent_hbm, rel_hbm, out_hbm,
             hidx, ridx, tidx, hbuf, rbuf, tbuf, sem_in, sem_out):
    wid = lax.axis_index("s") * _NUM_CORES + lax.axis_index("c")
    base = wid * _BPW

    for j in range(_NCHUNK):
        off = base + j * _CH
        pltpu.sync_copy(h_hbm.at[pl.ds(off, _CH)], hidx.at[j])
        pltpu.sync_copy(r_hbm.at[pl.ds(off, _CH)], ridx.at[j])
        pltpu.sync_copy(t_hbm.at[pl.ds(off, _CH)], tidx.at[j])

    def fire(j):
        s = j % 2
        copies = (
            pltpu.async_copy(ent_hbm.at[hidx.at[j]], hbuf.at[s], sem_in),
            pltpu.async_copy(ent_hbm.at[tidx.at[j]], tbuf.at[s], sem_in),
            pltpu.async_copy(rel_hbm.at[ridx.at[j]], rbuf.at[s], sem_in),
        )
        return copies

    in_flight = [fire(0)]
    out_flight = [None, None]
    for j in range(_NCHUNK):
        s = j % 2
        if j + 1 < _NCHUNK:
            if out_flight[(j + 1) % 2] is not None:
                out_flight[(j + 1) % 2].wait()
                out_flight[(j + 1) % 2] = None
            in_flight.append(fire(j + 1))
        for c in in_flight.pop(0):
            c.wait()

        def row(i, _):
            for u in range(2):
                for c in range(_DIM // _LANES):
                    sl = pl.ds(c * _LANES, _LANES)
                    hbuf[s, 2 * i + u, sl] = (
                        hbuf[s, 2 * i + u, sl]
                        + rbuf[s, 2 * i + u, sl]
                        - tbuf[s, 2 * i + u, sl]
                    )
            return 0

        lax.fori_loop(0, _CH // 2, row, 0)
        out_flight[s] = pltpu.async_copy(
            hbuf.at[s], out_hbm.at[pl.ds(base + j * _CH, _CH)], sem_out)

    for s in range(2):
        if out_flight[s] is not None:
            out_flight[s].wait()


_trans_e = functools.partial(
    pl.kernel,
    mesh=plsc.VectorSubcoreMesh(core_axis_name="c", subcore_axis_name="s"),
    out_type=jax.ShapeDtypeStruct((_BATCH, _DIM), jnp.float32),
    scratch_types=[
        pltpu.VMEM((_NCHUNK, _CH), jnp.int32),
        pltpu.VMEM((_NCHUNK, _CH), jnp.int32),
        pltpu.VMEM((_NCHUNK, _CH), jnp.int32),
        pltpu.VMEM((2, _CH, _DIM), jnp.float32),
        pltpu.VMEM((2, _CH, _DIM), jnp.float32),
        pltpu.VMEM((2, _CH, _DIM), jnp.float32),
        pltpu.SemaphoreType.DMA,
        pltpu.SemaphoreType.DMA,
    ],
    compiler_params=pltpu.CompilerParams(use_tc_tiling_on_sc=False),
)(_sc_body)


@jax.jit
def kernel(h, r, t, entity_embeddings, relation_embeddings):
    return _trans_e(
        h.astype(jnp.int32),
        r.astype(jnp.int32),
        t.astype(jnp.int32),
        entity_embeddings,
        relation_embeddings,
    )
